# baseline (speedup 1.0000x reference)
import functools

import jax
import jax.numpy as jnp
from jax import lax
from jax.experimental import pallas as pl
from jax.experimental.pallas import tpu as pltpu
from jax.experimental.pallas import tpu_sc as plsc

EMBED_DIM = 64
IN_DIM = 192
HIDDEN = 128
L = 16
SLAB = 128


_BCAST_DNUMS = lax.GatherDimensionNumbers(
    offset_dims=(), collapsed_slice_dims=(0,), start_index_map=(0,))


def _bcast_lane(ev, l):
    idx = jnp.full((L, 1), l, dtype=jnp.int32)
    return lax.gather(ev, idx, _BCAST_DNUMS, (1,),
                      mode=lax.GatherScatterMode.PROMISE_IN_BOUNDS)


def _xlane_sum(s):
    lane = lax.iota(jnp.int32, L)
    for sh in (8, 4, 2, 1):
        idx = ((lane + sh) & (L - 1)).reshape(L, 1)
        s = s + lax.gather(s, idx, _BCAST_DNUMS, (1,),
                           mode=lax.GatherScatterMode.PROMISE_IN_BOUNDS)
    return s


def _worker_id():
    return lax.axis_index("s") * 2 + lax.axis_index("c")


def _gather16(ref, rows, cols):
    return plsc.load_gather(ref, [rows, cols])


def _sc_body(speaker_h, word0_h, word1_h, t0T_h, t1T_h, stT_h,
             w1_h, b1_h, w2_h, b2_h, out_h,
             spk_i, w0_i, w1_i, s0_v, s1_v, s2_v, w1_v, b1_v, w2_v, b2_v,
             out_v, sem_idx, sem_g, sem_w):
    wid = _worker_id()

    @pl.when(wid == 0)
    def _():
        idx_cp = [
            pltpu.make_async_copy(speaker_h, spk_i.at[pl.ds(0, 1)], sem_idx),
            pltpu.make_async_copy(word0_h, w0_i.at[pl.ds(0, 1)], sem_idx),
            pltpu.make_async_copy(word1_h, w1_i.at[pl.ds(0, 1)], sem_idx),
        ]
        for c in idx_cp:
            c.start()
        w_cp = [
            pltpu.make_async_copy(w1_h, w1_v, sem_w),
            pltpu.make_async_copy(b1_h, b1_v, sem_w),
            pltpu.make_async_copy(w2_h, w2_v, sem_w),
        ]
        for c in w_cp:
            c.start()
        b2_v[...] = jnp.zeros((L,), jnp.float32)
        b2_cp = pltpu.make_async_copy(b2_h, b2_v.at[pl.ds(0, 1)], sem_w)
        b2_cp.start()
        for c in idx_cp:
            c.wait()

        iv = [spk_i[...], w0_i[...], w1_i[...]]
        bases = [pl.multiple_of((r[0] >> 7) << 7, SLAB) for r in iv]
        g_cp = [
            pltpu.make_async_copy(stT_h.at[:, pl.ds(bases[0], SLAB)],
                                  s0_v, sem_g),
            pltpu.make_async_copy(t0T_h.at[:, pl.ds(bases[1], SLAB)],
                                  s1_v, sem_g),
            pltpu.make_async_copy(t1T_h.at[:, pl.ds(bases[2], SLAB)],
                                  s2_v, sem_g),
        ]
        for c in g_cp:
            c.start()
        cols = [_bcast_lane(r, 0) & (SLAB - 1) for r in iv]
        for c in w_cp:
            c.wait()
        b2_cp.wait()
        for c in g_cp:
            c.wait()

        acc = [b1_v[pl.ds(16 * j, L)] for j in range(HIDDEN // L)]
        lane = lax.iota(jnp.int32, L)
        for r, slab_v in enumerate((s0_v, s1_v, s2_v)):
            for k in range(EMBED_DIM // L):
                ev = _gather16(slab_v, lane + 16 * k, cols[r])
                for l in range(L):
                    d = r * EMBED_DIM + k * L + l
                    eb = _bcast_lane(ev, l)
                    for j in range(HIDDEN // L):
                        acc[j] = acc[j] + eb * w1_v[d, pl.ds(16 * j, L)]

        s = jnp.zeros((L,), jnp.float32)
        for j in range(HIDDEN // L):
            h = jnp.maximum(acc[j], 0.0)
            s = s + h * w2_v[pl.ds(16 * j, L)]
        logit = _xlane_sum(s) + b2_v[...]
        out_v[...] = 1.0 / (1.0 + jnp.exp(-logit))
        pltpu.sync_copy(out_v, out_h)


@jax.jit
def _run(speaker, word0, word1, t0T, t1T, stT, W1, b1, W2r, b2):
    mesh = plsc.VectorSubcoreMesh(core_axis_name="c", subcore_axis_name="s",
                                  num_cores=2, num_subcores=16)
    f = pl.kernel(
        _sc_body,
        out_type=jax.ShapeDtypeStruct((L,), jnp.float32),
        mesh=mesh,
        scratch_types=[
            pltpu.VMEM((L,), jnp.int32),
            pltpu.VMEM((L,), jnp.int32),
            pltpu.VMEM((L,), jnp.int32),
            pltpu.VMEM((EMBED_DIM, SLAB), jnp.float32),
            pltpu.VMEM((EMBED_DIM, SLAB), jnp.float32),
            pltpu.VMEM((EMBED_DIM, SLAB), jnp.float32),
            pltpu.VMEM((IN_DIM, HIDDEN), jnp.float32),
            pltpu.VMEM((HIDDEN,), jnp.float32),
            pltpu.VMEM((HIDDEN,), jnp.float32),
            pltpu.VMEM((L,), jnp.float32),
            pltpu.VMEM((L,), jnp.float32),
            pltpu.SemaphoreType.DMA,
            pltpu.SemaphoreType.DMA,
            pltpu.SemaphoreType.DMA,
        ],
        compiler_params=pltpu.CompilerParams(needs_layout_passes=False),
    )
    return f(speaker, word0, word1, t0T, t1T, stT, W1, b1, W2r, b2)


def kernel(speaker, word0, word1, table0, table1, speaker_table, W1, b1, W2, b2):
    res = _run(speaker, word0, word1, table0.T, table1.T, speaker_table.T,
               W1, b1, W2.reshape(HIDDEN), b2)
    return res[0:1].reshape(1, 1)
